# baseline (speedup 1.0000x reference)
import functools

import jax
import jax.numpy as jnp
from jax import lax
from jax.experimental import pallas as pl
from jax.experimental.pallas import tpu as pltpu
from jax.experimental.pallas import tpu_sc as plsc

NN = 100000
EE = 3200000
NC, NS, LL = 2, 16, 16
NW = NC * NS
BKT_SHIFT = 7
BKT_N = 128
NB = (NN + BKT_N - 1) // BKT_N
NBP = 800
N_PAD = NB * BKT_N
EC = EE // NW
EBP = EE + NBP * 64 + 4096
DUMP = EBP - 1
ROW1, ROW2 = 384, 128
NEG = 0.2

_mesh = lambda: plsc.VectorSubcoreMesh(core_axis_name="c", subcore_axis_name="s")
_SC_PARAMS = pltpu.CompilerParams(needs_layout_passes=False)


def _wid():
    return lax.axis_index("s") * NC + lax.axis_index("c")


def _splat(vec, lane):
    return jnp.broadcast_to(vec[lane], (16,))


HCH = 10000


@functools.partial(
    pl.kernel,
    out_type=jax.ShapeDtypeStruct((NW, NBP * 16), jnp.int32),
    mesh=_mesh(),
    compiler_params=_SC_PARAMS,
    scratch_types=[pltpu.VMEM((HCH,), jnp.int32),
                   pltpu.VMEM((NBP * 16,), jnp.int32)],
)
def _hist_k(dst, counts_out, dbuf, cnt):
    w = _wid()
    base = pl.multiple_of(w * EC, 8)
    zero16 = jnp.zeros((16,), jnp.int32)

    def z(i, _):
        cnt[pl.ds(i * 16, 16)] = zero16
        return 0
    lax.fori_loop(0, NBP, z, 0)

    iota = lax.iota(jnp.int32, 16)
    one = jnp.ones((16,), jnp.int32)

    def chunk(ci, _):
        pltpu.sync_copy(dst.at[pl.ds(base + ci * HCH, HCH)], dbuf)

        def vec(vi, _):
            d = dbuf[pl.ds(vi * 16, 16)]
            ix = ((d >> BKT_SHIFT) << 4) | iota
            c = plsc.load_gather(cnt, [ix])
            plsc.store_scatter(cnt, [ix], c + one)
            return 0
        lax.fori_loop(0, HCH // 16, vec, 0)
        return 0
    lax.fori_loop(0, EC // HCH, chunk, 0)
    pltpu.sync_copy(cnt, counts_out.at[w])


def _prefix_body(cnt_ref, tri512_ref, tri400_ref, offs_ref, bs_ref, bc_ref):
    c = cnt_ref[...].reshape(NW, NBP, 16)
    c2 = jnp.concatenate([c[t] for t in range(NW)], axis=1)
    c2f = c2.astype(jnp.float32)
    inc = jnp.dot(c2f, tri512_ref[...],
                  preferred_element_type=jnp.float32).astype(jnp.int32)
    tot = inc[:, -1]
    sub = inc - c2
    reg = ((tot + 63) >> 6) << 6
    sinc = jnp.dot(reg.reshape(1, NBP).astype(jnp.float32), tri400_ref[...],
                   preferred_element_type=jnp.float32
                   ).astype(jnp.int32).reshape(NBP)
    starts = sinc - reg
    offs2 = starts[:, None] + sub
    offs = jnp.stack([offs2[:, t * 16:(t + 1) * 16] for t in range(NW)],
                     axis=0)
    offs_ref[...] = offs.reshape(NW, NBP * 16)
    bs_ref[...] = starts.reshape(1, NBP)
    bc_ref[...] = tot.reshape(1, NBP)


def _prefix(counts):
    tri512 = (jnp.arange(512)[:, None] <= jnp.arange(512)[None, :]
              ).astype(jnp.float32)
    tri400 = (jnp.arange(NBP)[:, None] <= jnp.arange(NBP)[None, :]
              ).astype(jnp.float32)
    return pl.pallas_call(
        _prefix_body,
        out_shape=(jax.ShapeDtypeStruct((NW, NBP * 16), jnp.int32),
                   jax.ShapeDtypeStruct((1, NBP), jnp.int32),
                   jax.ShapeDtypeStruct((1, NBP), jnp.int32)),
    )(counts, tri512, tri400)


SCH = 4096
SCH_T = EC - (EC // SCH) * SCH
SROWS = SCH // 128


@functools.partial(
    pl.kernel,
    out_type=jax.ShapeDtypeStruct((EBP,), jnp.int32),
    mesh=_mesh(),
    compiler_params=_SC_PARAMS,
    scratch_types=[pltpu.VMEM((SCH,), jnp.int32),
                   pltpu.VMEM((SCH,), jnp.int32),
                   pltpu.VMEM((SROWS, 128), jnp.int32),
                   pltpu.VMEM((SROWS, 128), jnp.int32),
                   pltpu.VMEM((NBP * 16,), jnp.int32),
                   pltpu.SemaphoreType.DMA],
)
def _scatter_k(src, dst, offs, pkb, sch, dch, posb, pkp, offl, sem0):
    w = _wid()
    base = pl.multiple_of(w * EC, 8)
    pltpu.sync_copy(offs.at[w], offl)
    iota = lax.iota(jnp.int32, 16)
    one = jnp.ones((16,), jnp.int32)
    dumpv = jnp.full((16,), DUMP, dtype=jnp.int32)

    def do_chunk(nvec):
        def vec(vi, _):
            r = vi >> 3
            m = vi & 7
            d = dch[pl.ds(vi * 16, 16)]
            sv = sch[pl.ds(vi * 16, 16)]
            ix = ((d >> BKT_SHIFT) << 4) | iota
            o = plsc.load_gather(offl, [ix])
            plsc.store_scatter(offl, [ix], o + one)
            posb[r, pl.ds(m * 16, 16)] = o
            pkp[r, pl.ds(m * 16, 16)] = (
                (sv << BKT_SHIFT) | (d & jnp.int32(BKT_N - 1)))
            return 0
        lax.fori_loop(0, nvec, vec, 0)
        for k in range(SROWS):
            pltpu.make_async_copy(pkp.at[k], pkb.at[posb.at[k]],
                                  sem0).start()
        for k in range(SROWS):
            pltpu.make_async_copy(pkp.at[k], pkb.at[posb.at[k]],
                                  sem0).wait()

    def chunk(ci, _):
        pltpu.sync_copy(src.at[pl.ds(base + ci * SCH, SCH)], sch)
        pltpu.sync_copy(dst.at[pl.ds(base + ci * SCH, SCH)], dch)
        do_chunk(SCH // 16)
        return 0
    lax.fori_loop(0, EC // SCH, chunk, 0)

    def fill(vi, _):
        r = vi >> 3
        m = vi & 7
        posb[r, pl.ds(m * 16, 16)] = dumpv
        return 0
    lax.fori_loop(0, SCH // 16, fill, 0)
    tbase = pl.multiple_of(base + (EC // SCH) * SCH, 8)
    pltpu.sync_copy(src.at[pl.ds(tbase, SCH_T)], sch.at[pl.ds(0, SCH_T)])
    pltpu.sync_copy(dst.at[pl.ds(tbase, SCH_T)], dch.at[pl.ds(0, SCH_T)])
    do_chunk(SCH_T // 16)


def _mm_body(x_ref, w_ref, o_ref):
    o_ref[...] = jnp.dot(x_ref[...], w_ref[...],
                         preferred_element_type=jnp.float32)


def _mm(x, w, n_rows, blk=BKT_N):
    k, m = w.shape
    grid = n_rows // blk
    return pl.pallas_call(
        _mm_body,
        grid=(grid,),
        in_specs=[pl.BlockSpec((blk, k), lambda i: (i, 0)),
                  pl.BlockSpec((k, m), lambda i: (0, 0))],
        out_specs=pl.BlockSpec((blk, m), lambda i: (i, 0)),
        out_shape=jax.ShapeDtypeStruct((n_rows, m), jnp.float32),
    )(x, w)


def _stats_body(x_ref, st_ref):
    blk = x_ref[...]
    s = jnp.sum(blk, axis=0, keepdims=True)
    s2 = jnp.sum(blk * blk, axis=0, keepdims=True)
    st = jnp.concatenate([s, s2], axis=0)

    @pl.when(pl.program_id(0) == 0)
    def _():
        st_ref[...] = st

    @pl.when(pl.program_id(0) > 0)
    def _():
        st_ref[...] = st_ref[...] + st


def _stats(x):
    n, m = x.shape
    return pl.pallas_call(
        _stats_body,
        grid=(n // BKT_N,),
        in_specs=[pl.BlockSpec((BKT_N, m), lambda i: (i, 0))],
        out_specs=pl.BlockSpec((2, m), lambda i: (0, 0)),
        out_shape=jax.ShapeDtypeStruct((2, m), jnp.float32),
    )(x)


def _bn_elu_mm_body(x_ref, mu_ref, isd_ref, g_ref, be_ref, w_ref, o_ref):
    xb = x_ref[...]
    y = g_ref[...] * (xb - mu_ref[...]) * isd_ref[...] + be_ref[...]
    y = jnp.where(y > 0, y, jnp.exp(jnp.minimum(y, 0.0)) - 1.0)
    o_ref[...] = jnp.dot(y, w_ref[...], preferred_element_type=jnp.float32)


def _bn_elu_mm(x, mu, isd, g, be, w):
    n, k = x.shape
    m = w.shape[1]
    return pl.pallas_call(
        _bn_elu_mm_body,
        grid=(n // BKT_N,),
        in_specs=[pl.BlockSpec((BKT_N, k), lambda i: (i, 0)),
                  pl.BlockSpec((1, k), lambda i: (0, 0)),
                  pl.BlockSpec((1, k), lambda i: (0, 0)),
                  pl.BlockSpec((1, k), lambda i: (0, 0)),
                  pl.BlockSpec((1, k), lambda i: (0, 0)),
                  pl.BlockSpec((k, m), lambda i: (0, 0))],
        out_specs=pl.BlockSpec((BKT_N, m), lambda i: (i, 0)),
        out_shape=jax.ShapeDtypeStruct((n, m), jnp.float32),
    )(x, mu.reshape(1, k), isd.reshape(1, k), g.reshape(1, k),
      be.reshape(1, k), w)


def _bn_elu_body(x_ref, mu_ref, isd_ref, g_ref, be_ref, o_ref):
    xb = x_ref[...]
    y = g_ref[...] * (xb - mu_ref[...]) * isd_ref[...] + be_ref[...]
    o_ref[...] = jnp.where(y > 0, y, jnp.exp(jnp.minimum(y, 0.0)) - 1.0)


def _bn_elu(x, mu, isd, g, be):
    n, k = x.shape
    return pl.pallas_call(
        _bn_elu_body,
        grid=(n // BKT_N,),
        in_specs=[pl.BlockSpec((BKT_N, k), lambda i: (i, 0)),
                  pl.BlockSpec((1, k), lambda i: (0, 0)),
                  pl.BlockSpec((1, k), lambda i: (0, 0)),
                  pl.BlockSpec((1, k), lambda i: (0, 0)),
                  pl.BlockSpec((1, k), lambda i: (0, 0))],
        out_specs=pl.BlockSpec((BKT_N, k), lambda i: (i, 0)),
        out_shape=jax.ShapeDtypeStruct((n, k), jnp.float32),
    )(x, mu.reshape(1, k), isd.reshape(1, k), g.reshape(1, k),
      be.reshape(1, k))


ACH = 1024


def _make_agg(row_w, heads, cw, gb):
    acc_w = heads * cw
    a_off = acc_w
    n_qh = cw // 16

    @functools.partial(
        pl.kernel,
        out_type=jax.ShapeDtypeStruct((N_PAD, acc_w), jnp.float32),
        mesh=_mesh(),
        compiler_params=_SC_PARAMS,
        scratch_types=[pltpu.VMEM((BKT_N, acc_w), jnp.float32),
                       pltpu.VMEM((BKT_N, 16), jnp.float32),
                       pltpu.VMEM((BKT_N, 16), jnp.float32),
                       pltpu.VMEM((2, gb, row_w), jnp.float32),
                       pltpu.VMEM((ACH,), jnp.int32),
                       pltpu.VMEM((ACH,), jnp.int32),
                       pltpu.VMEM((1, NBP), jnp.int32),
                       pltpu.VMEM((1, NBP), jnp.int32),
                       pltpu.SemaphoreType.DMA],
    )
    def agg(tbl, dvals, pkb, bs, bc, out, acc, den, dst_a, rows,
            pkc, sidx, bsv, bcv, sem):
        w = _wid()
        pltpu.sync_copy(bs, bsv)
        pltpu.sync_copy(bc, bcv)
        zf = jnp.zeros((16,), jnp.float32)
        eps = jnp.full((16,), 1e-16, dtype=jnp.float32)
        nmax = jnp.full((16,), NN - 1, dtype=jnp.int32)
        zi = jnp.zeros((16,), jnp.int32)
        dmask = jnp.full((16,), BKT_N - 1, dtype=jnp.int32)

        def bucket(bi, _):
            b = bi * NW + w

            @pl.when(b < NB)
            def _():
                nb = bcv[0, pl.ds(b, 16)][0]
                start = pl.multiple_of(bsv[0, pl.ds(b, 16)][0], 8)
                nbase = pl.multiple_of(b * BKT_N, 8)

                @plsc.parallel_loop(0, BKT_N, unroll=4)
                def zrow(r):
                    den[r, pl.ds(0, 16)] = zf
                    dst_a[r, pl.ds(0, 16)] = zf
                    for q in range(acc_w // 16):
                        acc[r, pl.ds(q * 16, 16)] = zf
                pltpu.sync_copy(dvals.at[pl.ds(nbase, BKT_N)], dst_a)

                nch = (nb + ACH - 1) // ACH

                def chunk(ci, _):
                    cbase = pl.multiple_of(start + ci * ACH, 8)
                    pltpu.sync_copy(pkb.at[pl.ds(cbase, ACH)], pkc)

                    @plsc.parallel_loop(0, ACH // 16, unroll=4)
                    def unpk(vi):
                        v = pkc[pl.ds(vi * 16, 16)]
                        sidx[pl.ds(vi * 16, 16)] = jnp.clip(
                            v >> BKT_SHIFT, zi, nmax)
                    nleft = nb - ci * ACH
                    nbat = jnp.minimum(
                        (nleft + gb - 1) // gb, ACH // gb)

                    def gref(k):
                        return sidx.at[pl.ds(k * gb, gb)]
                    pltpu.make_async_copy(tbl.at[gref(0)], rows.at[0],
                                          sem).start()

                    def batch(kb, _):
                        buf = kb & 1

                        @pl.when(kb + 1 < nbat)
                        def _():
                            pltpu.make_async_copy(
                                tbl.at[gref(kb + 1)],
                                rows.at[(kb + 1) & 1], sem).start()
                        pltpu.make_async_copy(tbl.at[gref(kb)],
                                              rows.at[buf], sem).wait()
                        ebase = kb * gb

                        @plsc.parallel_loop(0, gb // 16, unroll=2)
                        def grp(g):
                            dlv = pkc[pl.ds(ebase + g * 16, 16)] & dmask
                            for j in range(16):
                                jb = g * 16 + j
                                dl = dlv[j]
                                valid = (ebase + jb) < nleft
                                dv = dst_a[dl, pl.ds(0, 16)]
                                sv = rows[buf, jb, pl.ds(a_off, 16)]
                                a = sv + dv
                                a = jnp.maximum(a, NEG * a)
                                ex = jnp.where(valid, jnp.exp(a), zf)
                                plsc.addupdate(den.at[dl], ex)
                                for h in range(heads):
                                    eh = _splat(ex, h)
                                    for q in range(n_qh):
                                        c0 = h * cw + q * 16
                                        hv = rows[buf, jb, pl.ds(c0, 16)]
                                        plsc.addupdate(
                                            acc.at[dl, pl.ds(c0, 16)],
                                            eh * hv)
                        return 0
                    lax.fori_loop(0, nbat, batch, 0)
                    return 0
                lax.fori_loop(0, nch, chunk, 0)

                @plsc.parallel_loop(0, BKT_N, unroll=2)
                def nrow(r):
                    dinv = 1.0 / (den[r, pl.ds(0, 16)] + eps)
                    for h in range(heads):
                        eh = _splat(dinv, h)
                        for q in range(n_qh):
                            c0 = h * cw + q * 16
                            acc[r, pl.ds(c0, 16)] = acc[r, pl.ds(c0, 16)] * eh
                pltpu.sync_copy(acc, out.at[pl.ds(nbase, BKT_N)])
            return 0
        lax.fori_loop(0, (NB + NW - 1) // NW, bucket, 0)

    return agg


_agg1 = _make_agg(ROW1, 4, 64, 64)
_agg2 = _make_agg(ROW2, 1, 32, 128)


def _head_mats(a_src, a_dst, heads, cw):
    eye = jnp.eye(heads, dtype=jnp.float32)
    m = (eye[:, None, :] * a_src[:, :, None]).reshape(heads * cw, heads)
    md = (eye[:, None, :] * a_dst[:, :, None]).reshape(heads * cw, heads)
    pad = jnp.zeros((heads * cw, 16 - heads), jnp.float32)
    return (jnp.concatenate([m, pad], axis=1),
            jnp.concatenate([md, pad], axis=1))


def kernel(x, edge_index, W1, a_src1, a_dst1, b1, g1, be1,
           W2, a_src2, a_dst2, b2, g2, be2):
    f32 = jnp.float32
    As1, Ad1 = _head_mats(a_src1, a_dst1, 4, 64)
    As2, Ad2 = _head_mats(a_src2, a_dst2, 1, 32)
    hi = jax.lax.Precision.HIGHEST
    Wt1 = jnp.concatenate(
        [W1, jnp.dot(W1, As1, precision=hi),
         jnp.zeros((22, ROW1 - 272), f32)], axis=1
    ).astype(f32)
    Wd1 = jnp.dot(W1, Ad1, precision=hi).astype(f32)
    Wt2 = jnp.concatenate(
        [W2, jnp.dot(W2, As2, precision=hi),
         jnp.zeros((256, ROW2 - 48), f32)], axis=1
    ).astype(f32)
    Wd2 = jnp.dot(W2, Ad2, precision=hi).astype(f32)

    src_e = edge_index[0]
    dst_e = edge_index[1]
    counts = _hist_k(dst_e)
    offs, bstart, bcount = _prefix(counts)
    pkb = _scatter_k(src_e, dst_e, offs)

    t1 = _mm(x, Wt1, N_PAD)
    d1 = _mm(x, Wd1, N_PAD)

    o1 = _agg1(t1, d1, pkb, bstart, bcount)

    st1 = _stats(o1)
    mu1 = st1[0] / NN
    var1 = st1[1] / NN - mu1 * mu1
    isd1 = 1.0 / jnp.sqrt(var1 + 1e-5)

    t2 = _bn_elu_mm(o1, mu1, isd1, g1, be1, Wt2)
    d2 = _bn_elu_mm(o1, mu1, isd1, g1, be1, Wd2)

    o2 = _agg2(t2, d2, pkb, bstart, bcount)

    st2 = _stats(o2)
    mu2 = st2[0] / NN
    var2 = st2[1] / NN - mu2 * mu2
    isd2 = 1.0 / jnp.sqrt(var2 + 1e-5)
    out = _bn_elu(o2, mu2, isd2, g2, be2)
    return out[:NN]


# speedup vs baseline: 1.4017x; 1.4017x over previous
import functools

import jax
import jax.numpy as jnp
from jax import lax
from jax.experimental import pallas as pl
from jax.experimental.pallas import tpu as pltpu
from jax.experimental.pallas import tpu_sc as plsc

NN = 100000
EE = 3200000
NC, NS, LL = 2, 16, 16
NW = NC * NS
BKT_SHIFT = 7
BKT_N = 128
NB = (NN + BKT_N - 1) // BKT_N
NBP = 800
N_PAD = NB * BKT_N
EC = EE // NW
EBP = EE + NBP * 64 + 4096
DUMP = EBP - 1
ROW1, ROW2 = 384, 128
NEG = 0.2

_mesh = lambda: plsc.VectorSubcoreMesh(core_axis_name="c", subcore_axis_name="s")
_SC_PARAMS = pltpu.CompilerParams(needs_layout_passes=False)


def _wid():
    return lax.axis_index("s") * NC + lax.axis_index("c")


_DNUMS = lax.GatherDimensionNumbers(offset_dims=(),
                                    collapsed_slice_dims=(0,),
                                    start_index_map=(0,))


def _splat(vec, lane):
    idx = jnp.full((16, 1), lane, dtype=jnp.int32)
    return lax.gather(vec, idx, _DNUMS, (1,),
                      mode=lax.GatherScatterMode.PROMISE_IN_BOUNDS)


HCH = 10000


@functools.partial(
    pl.kernel,
    out_type=jax.ShapeDtypeStruct((NW, NBP * 16), jnp.int32),
    mesh=_mesh(),
    compiler_params=_SC_PARAMS,
    scratch_types=[pltpu.VMEM((HCH,), jnp.int32),
                   pltpu.VMEM((NBP * 16,), jnp.int32)],
)
def _hist_k(dst, counts_out, dbuf, cnt):
    w = _wid()
    base = pl.multiple_of(w * EC, 8)
    zero16 = jnp.zeros((16,), jnp.int32)

    def z(i, _):
        cnt[pl.ds(i * 16, 16)] = zero16
        return 0
    lax.fori_loop(0, NBP, z, 0)

    iota = lax.iota(jnp.int32, 16)
    one = jnp.ones((16,), jnp.int32)

    def chunk(ci, _):
        pltpu.sync_copy(dst.at[pl.ds(base + ci * HCH, HCH)], dbuf)

        def vec(vi, _):
            d = dbuf[pl.ds(vi * 16, 16)]
            ix = ((d >> BKT_SHIFT) << 4) | iota
            c = plsc.load_gather(cnt, [ix])
            plsc.store_scatter(cnt, [ix], c + one)
            return 0
        lax.fori_loop(0, HCH // 16, vec, 0)
        return 0
    lax.fori_loop(0, EC // HCH, chunk, 0)
    pltpu.sync_copy(cnt, counts_out.at[w])


def _prefix_body(cnt_ref, tri512_ref, tri400_ref, offs_ref, bs_ref, bc_ref):
    c = cnt_ref[...].reshape(NW, NBP, 16)
    c2 = jnp.concatenate([c[t] for t in range(NW)], axis=1)
    c2f = c2.astype(jnp.float32)
    inc = jnp.dot(c2f, tri512_ref[...],
                  preferred_element_type=jnp.float32).astype(jnp.int32)
    tot = inc[:, -1]
    sub = inc - c2
    reg = ((tot + 63) >> 6) << 6
    sinc = jnp.dot(reg.reshape(1, NBP).astype(jnp.float32), tri400_ref[...],
                   preferred_element_type=jnp.float32
                   ).astype(jnp.int32).reshape(NBP)
    starts = sinc - reg
    offs2 = starts[:, None] + sub
    offs = jnp.stack([offs2[:, t * 16:(t + 1) * 16] for t in range(NW)],
                     axis=0)
    offs_ref[...] = offs.reshape(NW, NBP * 16)
    bs_ref[...] = starts.reshape(1, NBP)
    bc_ref[...] = tot.reshape(1, NBP)


def _prefix(counts):
    tri512 = (jnp.arange(512)[:, None] <= jnp.arange(512)[None, :]
              ).astype(jnp.float32)
    tri400 = (jnp.arange(NBP)[:, None] <= jnp.arange(NBP)[None, :]
              ).astype(jnp.float32)
    return pl.pallas_call(
        _prefix_body,
        out_shape=(jax.ShapeDtypeStruct((NW, NBP * 16), jnp.int32),
                   jax.ShapeDtypeStruct((1, NBP), jnp.int32),
                   jax.ShapeDtypeStruct((1, NBP), jnp.int32)),
    )(counts, tri512, tri400)


SCH = 1024
SCH_T = EC - (EC // SCH) * SCH
SROWS = SCH // 128


@functools.partial(
    pl.kernel,
    out_type=jax.ShapeDtypeStruct((EBP,), jnp.int32),
    mesh=_mesh(),
    compiler_params=_SC_PARAMS,
    scratch_types=[pltpu.VMEM((SCH,), jnp.int32),
                   pltpu.VMEM((SCH,), jnp.int32),
                   pltpu.VMEM((SROWS, 128), jnp.int32),
                   pltpu.VMEM((SROWS, 128), jnp.int32),
                   pltpu.VMEM((NBP * 16,), jnp.int32),
                   pltpu.SemaphoreType.DMA],
)
def _scatter_k(src, dst, offs, pkb, sch, dch, posb, pkp, offl, sem0):
    w = _wid()
    base = pl.multiple_of(w * EC, 8)
    pltpu.sync_copy(offs.at[w], offl)
    iota = lax.iota(jnp.int32, 16)
    one = jnp.ones((16,), jnp.int32)
    dumpv = jnp.full((16,), DUMP, dtype=jnp.int32)

    def do_chunk(nvec):
        def vec(vi, _):
            r = vi >> 3
            m = vi & 7
            d = dch[pl.ds(vi * 16, 16)]
            sv = sch[pl.ds(vi * 16, 16)]
            ix = ((d >> BKT_SHIFT) << 4) | iota
            o = plsc.load_gather(offl, [ix])
            plsc.store_scatter(offl, [ix], o + one)
            posb[r, pl.ds(m * 16, 16)] = o
            pkp[r, pl.ds(m * 16, 16)] = (
                (sv << BKT_SHIFT) | (d & jnp.int32(BKT_N - 1)))
            return 0
        lax.fori_loop(0, nvec, vec, 0)
        for k in range(SROWS):
            pltpu.make_async_copy(pkp.at[k], pkb.at[posb.at[k]],
                                  sem0).start()
        for k in range(SROWS):
            pltpu.make_async_copy(pkp.at[k], pkb.at[posb.at[k]],
                                  sem0).wait()

    def chunk(ci, _):
        pltpu.sync_copy(src.at[pl.ds(base + ci * SCH, SCH)], sch)
        pltpu.sync_copy(dst.at[pl.ds(base + ci * SCH, SCH)], dch)
        do_chunk(SCH // 16)
        return 0
    lax.fori_loop(0, EC // SCH, chunk, 0)

    def fill(vi, _):
        r = vi >> 3
        m = vi & 7
        posb[r, pl.ds(m * 16, 16)] = dumpv
        return 0
    lax.fori_loop(0, SCH // 16, fill, 0)
    tbase = pl.multiple_of(base + (EC // SCH) * SCH, 8)
    pltpu.sync_copy(src.at[pl.ds(tbase, SCH_T)], sch.at[pl.ds(0, SCH_T)])
    pltpu.sync_copy(dst.at[pl.ds(tbase, SCH_T)], dch.at[pl.ds(0, SCH_T)])
    do_chunk(SCH_T // 16)


def _mm_body(x_ref, w_ref, o_ref):
    o_ref[...] = jnp.dot(x_ref[...], w_ref[...],
                         preferred_element_type=jnp.float32)


def _mm(x, w, n_rows, blk=BKT_N):
    k, m = w.shape
    grid = n_rows // blk
    return pl.pallas_call(
        _mm_body,
        grid=(grid,),
        in_specs=[pl.BlockSpec((blk, k), lambda i: (i, 0)),
                  pl.BlockSpec((k, m), lambda i: (0, 0))],
        out_specs=pl.BlockSpec((blk, m), lambda i: (i, 0)),
        out_shape=jax.ShapeDtypeStruct((n_rows, m), jnp.float32),
    )(x, w)


def _stats_body(x_ref, st_ref):
    blk = x_ref[...]
    s = jnp.sum(blk, axis=0, keepdims=True)
    s2 = jnp.sum(blk * blk, axis=0, keepdims=True)
    st = jnp.concatenate([s, s2], axis=0)

    @pl.when(pl.program_id(0) == 0)
    def _():
        st_ref[...] = st

    @pl.when(pl.program_id(0) > 0)
    def _():
        st_ref[...] = st_ref[...] + st


def _stats(x):
    n, m = x.shape
    return pl.pallas_call(
        _stats_body,
        grid=(n // BKT_N,),
        in_specs=[pl.BlockSpec((BKT_N, m), lambda i: (i, 0))],
        out_specs=pl.BlockSpec((2, m), lambda i: (0, 0)),
        out_shape=jax.ShapeDtypeStruct((2, m), jnp.float32),
    )(x)


def _bn_elu_mm_body(x_ref, mu_ref, isd_ref, g_ref, be_ref, w_ref, o_ref):
    xb = x_ref[...]
    y = g_ref[...] * (xb - mu_ref[...]) * isd_ref[...] + be_ref[...]
    y = jnp.where(y > 0, y, jnp.exp(jnp.minimum(y, 0.0)) - 1.0)
    o_ref[...] = jnp.dot(y, w_ref[...], preferred_element_type=jnp.float32)


def _bn_elu_mm(x, mu, isd, g, be, w):
    n, k = x.shape
    m = w.shape[1]
    return pl.pallas_call(
        _bn_elu_mm_body,
        grid=(n // BKT_N,),
        in_specs=[pl.BlockSpec((BKT_N, k), lambda i: (i, 0)),
                  pl.BlockSpec((1, k), lambda i: (0, 0)),
                  pl.BlockSpec((1, k), lambda i: (0, 0)),
                  pl.BlockSpec((1, k), lambda i: (0, 0)),
                  pl.BlockSpec((1, k), lambda i: (0, 0)),
                  pl.BlockSpec((k, m), lambda i: (0, 0))],
        out_specs=pl.BlockSpec((BKT_N, m), lambda i: (i, 0)),
        out_shape=jax.ShapeDtypeStruct((n, m), jnp.float32),
    )(x, mu.reshape(1, k), isd.reshape(1, k), g.reshape(1, k),
      be.reshape(1, k), w)


def _bn_elu_body(x_ref, mu_ref, isd_ref, g_ref, be_ref, o_ref):
    xb = x_ref[...]
    y = g_ref[...] * (xb - mu_ref[...]) * isd_ref[...] + be_ref[...]
    o_ref[...] = jnp.where(y > 0, y, jnp.exp(jnp.minimum(y, 0.0)) - 1.0)


def _bn_elu(x, mu, isd, g, be):
    n, k = x.shape
    return pl.pallas_call(
        _bn_elu_body,
        grid=(n // BKT_N,),
        in_specs=[pl.BlockSpec((BKT_N, k), lambda i: (i, 0)),
                  pl.BlockSpec((1, k), lambda i: (0, 0)),
                  pl.BlockSpec((1, k), lambda i: (0, 0)),
                  pl.BlockSpec((1, k), lambda i: (0, 0)),
                  pl.BlockSpec((1, k), lambda i: (0, 0))],
        out_specs=pl.BlockSpec((BKT_N, k), lambda i: (i, 0)),
        out_shape=jax.ShapeDtypeStruct((n, k), jnp.float32),
    )(x, mu.reshape(1, k), isd.reshape(1, k), g.reshape(1, k),
      be.reshape(1, k))


ACH = 1024


def _make_agg(row_w, heads, cw, gb):
    acc_w = heads * cw
    a_off = acc_w
    n_qh = cw // 16

    @functools.partial(
        pl.kernel,
        out_type=jax.ShapeDtypeStruct((N_PAD, acc_w), jnp.float32),
        mesh=_mesh(),
        compiler_params=_SC_PARAMS,
        scratch_types=[pltpu.VMEM((BKT_N, acc_w), jnp.float32),
                       pltpu.VMEM((BKT_N, 16), jnp.float32),
                       pltpu.VMEM((BKT_N, 16), jnp.float32),
                       pltpu.VMEM((2, gb, row_w), jnp.float32),
                       pltpu.VMEM((ACH,), jnp.int32),
                       pltpu.VMEM((ACH,), jnp.int32),
                       pltpu.VMEM((1, NBP), jnp.int32),
                       pltpu.VMEM((1, NBP), jnp.int32),
                       pltpu.SemaphoreType.DMA],
    )
    def agg(tbl, dvals, pkb, bs, bc, out, acc, den, dst_a, rows,
            pkc, sidx, bsv, bcv, sem):
        w = _wid()
        pltpu.sync_copy(bs, bsv)
        pltpu.sync_copy(bc, bcv)
        zf = jnp.zeros((16,), jnp.float32)
        eps = jnp.full((16,), 1e-16, dtype=jnp.float32)
        nmax = jnp.full((16,), NN - 1, dtype=jnp.int32)
        zi = jnp.zeros((16,), jnp.int32)
        dmask = jnp.full((16,), BKT_N - 1, dtype=jnp.int32)

        def bucket(bi, _):
            b = bi * NW + w

            @pl.when(b < NB)
            def _():
                nb = bcv[0, pl.ds(b, 16)][0]
                start = pl.multiple_of(bsv[0, pl.ds(b, 16)][0], 8)
                nbase = pl.multiple_of(b * BKT_N, 8)

                @plsc.parallel_loop(0, BKT_N, unroll=4)
                def zrow(r):
                    den[r, pl.ds(0, 16)] = zf
                    dst_a[r, pl.ds(0, 16)] = zf
                    for q in range(acc_w // 16):
                        acc[r, pl.ds(q * 16, 16)] = zf
                pltpu.sync_copy(dvals.at[pl.ds(nbase, BKT_N)], dst_a)

                nch = (nb + ACH - 1) // ACH

                def chunk(ci, _):
                    cbase = pl.multiple_of(start + ci * ACH, 8)
                    pltpu.sync_copy(pkb.at[pl.ds(cbase, ACH)], pkc)

                    @plsc.parallel_loop(0, ACH // 16, unroll=4)
                    def unpk(vi):
                        v = pkc[pl.ds(vi * 16, 16)]
                        sidx[pl.ds(vi * 16, 16)] = jnp.clip(
                            v >> BKT_SHIFT, zi, nmax)
                    nleft = nb - ci * ACH
                    nbat = jnp.minimum(
                        (nleft + gb - 1) // gb, ACH // gb)

                    def gref(k):
                        return sidx.at[pl.ds(k * gb, gb)]
                    pltpu.make_async_copy(tbl.at[gref(0)], rows.at[0],
                                          sem).start()

                    def batch(kb, _):
                        buf = kb & 1

                        @pl.when(kb + 1 < nbat)
                        def _():
                            pltpu.make_async_copy(
                                tbl.at[gref(kb + 1)],
                                rows.at[(kb + 1) & 1], sem).start()
                        pltpu.make_async_copy(tbl.at[gref(kb)],
                                              rows.at[buf], sem).wait()
                        ebase = kb * gb

                        @plsc.parallel_loop(0, gb // 16, unroll=2)
                        def grp(g):
                            dlv = pkc[pl.ds(ebase + g * 16, 16)] & dmask
                            for j in range(16):
                                jb = g * 16 + j
                                dl = dlv[j]
                                valid = (ebase + jb) < nleft
                                dv = dst_a[dl, pl.ds(0, 16)]
                                sv = rows[buf, jb, pl.ds(a_off, 16)]
                                a = sv + dv
                                a = jnp.maximum(a, NEG * a)
                                ex = jnp.where(valid, jnp.exp(a), zf)
                                plsc.addupdate(den.at[dl], ex)
                                for h in range(heads):
                                    eh = _splat(ex, h)
                                    for q in range(n_qh):
                                        c0 = h * cw + q * 16
                                        hv = rows[buf, jb, pl.ds(c0, 16)]
                                        plsc.addupdate(
                                            acc.at[dl, pl.ds(c0, 16)],
                                            eh * hv)
                        return 0
                    lax.fori_loop(0, nbat, batch, 0)
                    return 0
                lax.fori_loop(0, nch, chunk, 0)

                @plsc.parallel_loop(0, BKT_N, unroll=2)
                def nrow(r):
                    dinv = 1.0 / (den[r, pl.ds(0, 16)] + eps)
                    for h in range(heads):
                        eh = _splat(dinv, h)
                        for q in range(n_qh):
                            c0 = h * cw + q * 16
                            acc[r, pl.ds(c0, 16)] = acc[r, pl.ds(c0, 16)] * eh
                pltpu.sync_copy(acc, out.at[pl.ds(nbase, BKT_N)])
            return 0
        lax.fori_loop(0, (NB + NW - 1) // NW, bucket, 0)

    return agg


_agg1 = _make_agg(ROW1, 4, 64, 64)
_agg2 = _make_agg(ROW2, 1, 32, 128)


def _head_mats(a_src, a_dst, heads, cw):
    eye = jnp.eye(heads, dtype=jnp.float32)
    m = (eye[:, None, :] * a_src[:, :, None]).reshape(heads * cw, heads)
    md = (eye[:, None, :] * a_dst[:, :, None]).reshape(heads * cw, heads)
    pad = jnp.zeros((heads * cw, 16 - heads), jnp.float32)
    return (jnp.concatenate([m, pad], axis=1),
            jnp.concatenate([md, pad], axis=1))


def kernel(x, edge_index, W1, a_src1, a_dst1, b1, g1, be1,
           W2, a_src2, a_dst2, b2, g2, be2):
    f32 = jnp.float32
    As1, Ad1 = _head_mats(a_src1, a_dst1, 4, 64)
    As2, Ad2 = _head_mats(a_src2, a_dst2, 1, 32)
    hi = jax.lax.Precision.HIGHEST
    Wt1 = jnp.concatenate(
        [W1, jnp.dot(W1, As1, precision=hi),
         jnp.zeros((22, ROW1 - 272), f32)], axis=1
    ).astype(f32)
    Wd1 = jnp.dot(W1, Ad1, precision=hi).astype(f32)
    Wt2 = jnp.concatenate(
        [W2, jnp.dot(W2, As2, precision=hi),
         jnp.zeros((256, ROW2 - 48), f32)], axis=1
    ).astype(f32)
    Wd2 = jnp.dot(W2, Ad2, precision=hi).astype(f32)

    src_e = edge_index[0]
    dst_e = edge_index[1]
    counts = _hist_k(dst_e)
    offs, bstart, bcount = _prefix(counts)
    pkb = _scatter_k(src_e, dst_e, offs)

    t1 = _mm(x, Wt1, N_PAD)
    d1 = _mm(x, Wd1, N_PAD)

    o1 = _agg1(t1, d1, pkb, bstart, bcount)

    st1 = _stats(o1)
    mu1 = st1[0] / NN
    var1 = st1[1] / NN - mu1 * mu1
    isd1 = 1.0 / jnp.sqrt(var1 + 1e-5)

    t2 = _bn_elu_mm(o1, mu1, isd1, g1, be1, Wt2)
    d2 = _bn_elu_mm(o1, mu1, isd1, g1, be1, Wd2)

    o2 = _agg2(t2, d2, pkb, bstart, bcount)

    st2 = _stats(o2)
    mu2 = st2[0] / NN
    var2 = st2[1] / NN - mu2 * mu2
    isd2 = 1.0 / jnp.sqrt(var2 + 1e-5)
    out = _bn_elu(o2, mu2, isd2, g2, be2)
    return out[:NN]


# speedup vs baseline: 2.8809x; 2.0553x over previous
import functools

import jax
import jax.numpy as jnp
from jax import lax
from jax.experimental import pallas as pl
from jax.experimental.pallas import tpu as pltpu
from jax.experimental.pallas import tpu_sc as plsc

NN = 100000
EE = 3200000
NC, NS, LL = 2, 16, 16
NW = NC * NS
BKT_SHIFT = 7
BKT_N = 128
NB = (NN + BKT_N - 1) // BKT_N
NBP = 800
N_PAD = NB * BKT_N
EC = EE // NW
EBP = EE + NBP * 64 + 4096
DUMP = EBP - 1
ROW1, ROW2 = 384, 128
NEG = 0.2

_mesh = lambda: plsc.VectorSubcoreMesh(core_axis_name="c", subcore_axis_name="s")
_SC_PARAMS = pltpu.CompilerParams(needs_layout_passes=False)


def _wid():
    return lax.axis_index("s") * NC + lax.axis_index("c")


_DNUMS = lax.GatherDimensionNumbers(offset_dims=(),
                                    collapsed_slice_dims=(0,),
                                    start_index_map=(0,))


def _splat(vec, lane):
    idx = jnp.full((16, 1), lane, dtype=jnp.int32)
    return lax.gather(vec, idx, _DNUMS, (1,),
                      mode=lax.GatherScatterMode.PROMISE_IN_BOUNDS)


HCH = 10000


@functools.partial(
    pl.kernel,
    out_type=jax.ShapeDtypeStruct((NW, NBP * 16), jnp.int32),
    mesh=_mesh(),
    compiler_params=_SC_PARAMS,
    scratch_types=[pltpu.VMEM((HCH,), jnp.int32),
                   pltpu.VMEM((NBP * 16,), jnp.int32)],
)
def _hist_k(dst, counts_out, dbuf, cnt):
    w = _wid()
    base = pl.multiple_of(w * EC, 8)
    zero16 = jnp.zeros((16,), jnp.int32)

    def z(i, _):
        cnt[pl.ds(i * 16, 16)] = zero16
        return 0
    lax.fori_loop(0, NBP, z, 0)

    iota = lax.iota(jnp.int32, 16)
    one = jnp.ones((16,), jnp.int32)

    def chunk(ci, _):
        pltpu.sync_copy(dst.at[pl.ds(base + ci * HCH, HCH)], dbuf)

        def vec(vi, _):
            d = dbuf[pl.ds(vi * 16, 16)]
            ix = ((d >> BKT_SHIFT) << 4) | iota
            c = plsc.load_gather(cnt, [ix])
            plsc.store_scatter(cnt, [ix], c + one)
            return 0
        lax.fori_loop(0, HCH // 16, vec, 0)
        return 0
    lax.fori_loop(0, EC // HCH, chunk, 0)
    pltpu.sync_copy(cnt, counts_out.at[w])


def _prefix_body(cnt_ref, tri512_ref, tri400_ref, offs_ref, bs_ref, bc_ref):
    c = cnt_ref[...].reshape(NW, NBP, 16)
    c2 = jnp.concatenate([c[t] for t in range(NW)], axis=1)
    c2f = c2.astype(jnp.float32)
    inc = jnp.dot(c2f, tri512_ref[...],
                  preferred_element_type=jnp.float32).astype(jnp.int32)
    tot = inc[:, -1]
    sub = inc - c2
    reg = ((tot + 63) >> 6) << 6
    sinc = jnp.dot(reg.reshape(1, NBP).astype(jnp.float32), tri400_ref[...],
                   preferred_element_type=jnp.float32
                   ).astype(jnp.int32).reshape(NBP)
    starts = sinc - reg
    offs2 = starts[:, None] + sub
    offs = jnp.stack([offs2[:, t * 16:(t + 1) * 16] for t in range(NW)],
                     axis=0)
    offs_ref[...] = offs.reshape(NW, NBP * 16)
    bs_ref[...] = starts.reshape(1, NBP)
    bc_ref[...] = tot.reshape(1, NBP)


def _prefix(counts):
    tri512 = (jnp.arange(512)[:, None] <= jnp.arange(512)[None, :]
              ).astype(jnp.float32)
    tri400 = (jnp.arange(NBP)[:, None] <= jnp.arange(NBP)[None, :]
              ).astype(jnp.float32)
    return pl.pallas_call(
        _prefix_body,
        out_shape=(jax.ShapeDtypeStruct((NW, NBP * 16), jnp.int32),
                   jax.ShapeDtypeStruct((1, NBP), jnp.int32),
                   jax.ShapeDtypeStruct((1, NBP), jnp.int32)),
    )(counts, tri512, tri400)


SCH = 1024
SCH_T = EC - (EC // SCH) * SCH
SROWS = SCH // 128


@functools.partial(
    pl.kernel,
    out_type=jax.ShapeDtypeStruct((EBP,), jnp.int32),
    mesh=_mesh(),
    compiler_params=_SC_PARAMS,
    scratch_types=[pltpu.VMEM((SCH,), jnp.int32),
                   pltpu.VMEM((SCH,), jnp.int32),
                   pltpu.VMEM((SROWS, 128), jnp.int32),
                   pltpu.VMEM((SROWS, 128), jnp.int32),
                   pltpu.VMEM((NBP * 16,), jnp.int32),
                   pltpu.SemaphoreType.DMA],
)
def _scatter_k(src, dst, offs, pkb, sch, dch, posb, pkp, offl, sem0):
    w = _wid()
    base = pl.multiple_of(w * EC, 8)
    pltpu.sync_copy(offs.at[w], offl)
    iota = lax.iota(jnp.int32, 16)
    one = jnp.ones((16,), jnp.int32)
    dumpv = jnp.full((16,), DUMP, dtype=jnp.int32)

    def do_chunk(nvec):
        def vec(vi, _):
            r = vi >> 3
            m = vi & 7
            d = dch[pl.ds(vi * 16, 16)]
            sv = sch[pl.ds(vi * 16, 16)]
            ix = ((d >> BKT_SHIFT) << 4) | iota
            o = plsc.load_gather(offl, [ix])
            plsc.store_scatter(offl, [ix], o + one)
            posb[r, pl.ds(m * 16, 16)] = o
            pkp[r, pl.ds(m * 16, 16)] = (
                (sv << BKT_SHIFT) | (d & jnp.int32(BKT_N - 1)))
            return 0
        lax.fori_loop(0, nvec, vec, 0)
        for k in range(SROWS):
            pltpu.make_async_copy(pkp.at[k], pkb.at[posb.at[k]],
                                  sem0).start()
        for k in range(SROWS):
            pltpu.make_async_copy(pkp.at[k], pkb.at[posb.at[k]],
                                  sem0).wait()

    def chunk(ci, _):
        pltpu.sync_copy(src.at[pl.ds(base + ci * SCH, SCH)], sch)
        pltpu.sync_copy(dst.at[pl.ds(base + ci * SCH, SCH)], dch)
        do_chunk(SCH // 16)
        return 0
    lax.fori_loop(0, EC // SCH, chunk, 0)

    def fill(vi, _):
        r = vi >> 3
        m = vi & 7
        posb[r, pl.ds(m * 16, 16)] = dumpv
        return 0
    lax.fori_loop(0, SCH // 16, fill, 0)
    tbase = pl.multiple_of(base + (EC // SCH) * SCH, 8)
    pltpu.sync_copy(src.at[pl.ds(tbase, SCH_T)], sch.at[pl.ds(0, SCH_T)])
    pltpu.sync_copy(dst.at[pl.ds(tbase, SCH_T)], dch.at[pl.ds(0, SCH_T)])
    do_chunk(SCH_T // 16)


def _mm_body(x_ref, w_ref, o_ref):
    o_ref[...] = jnp.dot(x_ref[...], w_ref[...],
                         preferred_element_type=jnp.float32)


def _mm(x, w, n_rows, blk=BKT_N):
    k, m = w.shape
    grid = n_rows // blk
    return pl.pallas_call(
        _mm_body,
        grid=(grid,),
        in_specs=[pl.BlockSpec((blk, k), lambda i: (i, 0)),
                  pl.BlockSpec((k, m), lambda i: (0, 0))],
        out_specs=pl.BlockSpec((blk, m), lambda i: (i, 0)),
        out_shape=jax.ShapeDtypeStruct((n_rows, m), jnp.float32),
    )(x, w)


def _stats_body(x_ref, st_ref):
    blk = x_ref[...]
    s = jnp.sum(blk, axis=0, keepdims=True)
    s2 = jnp.sum(blk * blk, axis=0, keepdims=True)
    st = jnp.concatenate([s, s2], axis=0)

    @pl.when(pl.program_id(0) == 0)
    def _():
        st_ref[...] = st

    @pl.when(pl.program_id(0) > 0)
    def _():
        st_ref[...] = st_ref[...] + st


def _stats(x):
    n, m = x.shape
    return pl.pallas_call(
        _stats_body,
        grid=(n // BKT_N,),
        in_specs=[pl.BlockSpec((BKT_N, m), lambda i: (i, 0))],
        out_specs=pl.BlockSpec((2, m), lambda i: (0, 0)),
        out_shape=jax.ShapeDtypeStruct((2, m), jnp.float32),
    )(x)


def _bn_elu_mm_body(x_ref, mu_ref, isd_ref, g_ref, be_ref, w_ref, o_ref):
    xb = x_ref[...]
    y = g_ref[...] * (xb - mu_ref[...]) * isd_ref[...] + be_ref[...]
    y = jnp.where(y > 0, y, jnp.exp(jnp.minimum(y, 0.0)) - 1.0)
    o_ref[...] = jnp.dot(y, w_ref[...], preferred_element_type=jnp.float32)


def _bn_elu_mm(x, mu, isd, g, be, w):
    n, k = x.shape
    m = w.shape[1]
    return pl.pallas_call(
        _bn_elu_mm_body,
        grid=(n // BKT_N,),
        in_specs=[pl.BlockSpec((BKT_N, k), lambda i: (i, 0)),
                  pl.BlockSpec((1, k), lambda i: (0, 0)),
                  pl.BlockSpec((1, k), lambda i: (0, 0)),
                  pl.BlockSpec((1, k), lambda i: (0, 0)),
                  pl.BlockSpec((1, k), lambda i: (0, 0)),
                  pl.BlockSpec((k, m), lambda i: (0, 0))],
        out_specs=pl.BlockSpec((BKT_N, m), lambda i: (i, 0)),
        out_shape=jax.ShapeDtypeStruct((n, m), jnp.float32),
    )(x, mu.reshape(1, k), isd.reshape(1, k), g.reshape(1, k),
      be.reshape(1, k), w)


def _bn_elu_body(x_ref, mu_ref, isd_ref, g_ref, be_ref, o_ref):
    xb = x_ref[...]
    y = g_ref[...] * (xb - mu_ref[...]) * isd_ref[...] + be_ref[...]
    o_ref[...] = jnp.where(y > 0, y, jnp.exp(jnp.minimum(y, 0.0)) - 1.0)


def _bn_elu(x, mu, isd, g, be):
    n, k = x.shape
    return pl.pallas_call(
        _bn_elu_body,
        grid=(n // BKT_N,),
        in_specs=[pl.BlockSpec((BKT_N, k), lambda i: (i, 0)),
                  pl.BlockSpec((1, k), lambda i: (0, 0)),
                  pl.BlockSpec((1, k), lambda i: (0, 0)),
                  pl.BlockSpec((1, k), lambda i: (0, 0)),
                  pl.BlockSpec((1, k), lambda i: (0, 0))],
        out_specs=pl.BlockSpec((BKT_N, k), lambda i: (i, 0)),
        out_shape=jax.ShapeDtypeStruct((n, k), jnp.float32),
    )(x, mu.reshape(1, k), isd.reshape(1, k), g.reshape(1, k),
      be.reshape(1, k))


ACH = 1024


def _make_agg(row_w, heads, cw, gb):
    acc_w = heads * cw
    a_off = acc_w
    n_qh = cw // 16

    @functools.partial(
        pl.kernel,
        out_type=jax.ShapeDtypeStruct((N_PAD, acc_w), jnp.float32),
        mesh=_mesh(),
        compiler_params=_SC_PARAMS,
        scratch_types=[pltpu.VMEM((BKT_N, acc_w), jnp.float32),
                       pltpu.VMEM((BKT_N, 16), jnp.float32),
                       pltpu.VMEM((BKT_N, 16), jnp.float32),
                       pltpu.VMEM((2, gb, row_w), jnp.float32),
                       pltpu.VMEM((ACH,), jnp.int32),
                       pltpu.VMEM((ACH,), jnp.int32),
                       pltpu.VMEM((1, NBP), jnp.int32),
                       pltpu.VMEM((1, NBP), jnp.int32),
                       pltpu.SemaphoreType.DMA],
    )
    def agg(tbl, dvals, pkb, bs, bc, out, acc, den, dst_a, rows,
            pkc, sidx, bsv, bcv, sem):
        w = _wid()
        pltpu.sync_copy(bs, bsv)
        pltpu.sync_copy(bc, bcv)
        zf = jnp.zeros((16,), jnp.float32)
        eps = jnp.full((16,), 1e-16, dtype=jnp.float32)
        nmax = jnp.full((16,), NN - 1, dtype=jnp.int32)
        zi = jnp.zeros((16,), jnp.int32)
        dmask = jnp.full((16,), BKT_N - 1, dtype=jnp.int32)

        def bucket(bi, _):
            b = bi * NW + w

            @pl.when(b < NB)
            def _():
                nb = bcv[0, pl.ds(b, 16)][0]
                start = pl.multiple_of(bsv[0, pl.ds(b, 16)][0], 8)
                nbase = pl.multiple_of(b * BKT_N, 8)

                @plsc.parallel_loop(0, BKT_N, unroll=4)
                def zrow(r):
                    den[r, pl.ds(0, 16)] = zf
                    dst_a[r, pl.ds(0, 16)] = zf
                    for q in range(acc_w // 16):
                        acc[r, pl.ds(q * 16, 16)] = zf
                pltpu.sync_copy(dvals.at[pl.ds(nbase, BKT_N)], dst_a)

                nch = (nb + ACH - 1) // ACH

                def chunk(ci, _):
                    cbase = pl.multiple_of(start + ci * ACH, 8)
                    pltpu.sync_copy(pkb.at[pl.ds(cbase, ACH)], pkc)

                    @plsc.parallel_loop(0, ACH // 16, unroll=4)
                    def unpk(vi):
                        v = pkc[pl.ds(vi * 16, 16)]
                        sidx[pl.ds(vi * 16, 16)] = jnp.clip(
                            v >> BKT_SHIFT, zi, nmax)
                    nleft = nb - ci * ACH
                    nbat = jnp.minimum(
                        (nleft + gb - 1) // gb, ACH // gb)

                    def gref(k):
                        return sidx.at[pl.ds(k * gb, gb)]
                    pltpu.make_async_copy(tbl.at[gref(0)], rows.at[0],
                                          sem).start()

                    def batch(kb, _):
                        buf = kb & 1

                        @pl.when(kb + 1 < nbat)
                        def _():
                            pltpu.make_async_copy(
                                tbl.at[gref(kb + 1)],
                                rows.at[(kb + 1) & 1], sem).start()
                        pltpu.make_async_copy(tbl.at[gref(kb)],
                                              rows.at[buf], sem).wait()
                        ebase = kb * gb

                        @plsc.parallel_loop(0, gb // 16, unroll=2)
                        def grp(g):
                            dlv = pkc[pl.ds(ebase + g * 16, 16)] & dmask
                            for j in range(16):
                                jb = g * 16 + j
                                dl = dlv[j]
                                valid = (ebase + jb) < nleft
                                dv = dst_a[dl, pl.ds(0, 16)]
                                sv = rows[buf, jb, pl.ds(a_off, 16)]
                                hvs = [rows[buf, jb,
                                            pl.ds(h * cw + q * 16, 16)]
                                       for h in range(heads)
                                       for q in range(n_qh)]
                                a = sv + dv
                                a = jnp.maximum(a, NEG * a)
                                ex = jnp.where(valid, jnp.exp(a), zf)
                                ehs = [_splat(ex, h) for h in range(heads)]
                                for h in range(heads):
                                    for q in range(n_qh):
                                        c0 = h * cw + q * 16
                                        plsc.addupdate(
                                            acc.at[dl, pl.ds(c0, 16)],
                                            ehs[h] * hvs[h * n_qh + q])
                                plsc.addupdate(den.at[dl], ex)
                        return 0
                    lax.fori_loop(0, nbat, batch, 0)
                    return 0
                lax.fori_loop(0, nch, chunk, 0)

                @plsc.parallel_loop(0, BKT_N, unroll=2)
                def nrow(r):
                    dinv = 1.0 / (den[r, pl.ds(0, 16)] + eps)
                    for h in range(heads):
                        eh = _splat(dinv, h)
                        for q in range(n_qh):
                            c0 = h * cw + q * 16
                            acc[r, pl.ds(c0, 16)] = acc[r, pl.ds(c0, 16)] * eh
                pltpu.sync_copy(acc, out.at[pl.ds(nbase, BKT_N)])
            return 0
        lax.fori_loop(0, (NB + NW - 1) // NW, bucket, 0)

    return agg


_agg1 = _make_agg(ROW1, 4, 64, 64)
_agg2 = _make_agg(ROW2, 1, 32, 128)


def _head_mats(a_src, a_dst, heads, cw):
    eye = jnp.eye(heads, dtype=jnp.float32)
    m = (eye[:, None, :] * a_src[:, :, None]).reshape(heads * cw, heads)
    md = (eye[:, None, :] * a_dst[:, :, None]).reshape(heads * cw, heads)
    pad = jnp.zeros((heads * cw, 16 - heads), jnp.float32)
    return (jnp.concatenate([m, pad], axis=1),
            jnp.concatenate([md, pad], axis=1))


def kernel(x, edge_index, W1, a_src1, a_dst1, b1, g1, be1,
           W2, a_src2, a_dst2, b2, g2, be2):
    f32 = jnp.float32
    As1, Ad1 = _head_mats(a_src1, a_dst1, 4, 64)
    As2, Ad2 = _head_mats(a_src2, a_dst2, 1, 32)
    hi = jax.lax.Precision.HIGHEST
    Wt1 = jnp.concatenate(
        [W1, jnp.dot(W1, As1, precision=hi),
         jnp.zeros((22, ROW1 - 272), f32)], axis=1
    ).astype(f32)
    Wd1 = jnp.dot(W1, Ad1, precision=hi).astype(f32)
    Wt2 = jnp.concatenate(
        [W2, jnp.dot(W2, As2, precision=hi),
         jnp.zeros((256, ROW2 - 48), f32)], axis=1
    ).astype(f32)
    Wd2 = jnp.dot(W2, Ad2, precision=hi).astype(f32)

    src_e = edge_index[0]
    dst_e = edge_index[1]
    counts = _hist_k(dst_e)
    offs, bstart, bcount = _prefix(counts)
    pkb = _scatter_k(src_e, dst_e, offs)

    t1 = _mm(x, Wt1, N_PAD)
    d1 = _mm(x, Wd1, N_PAD)

    o1 = _agg1(t1, d1, pkb, bstart, bcount)

    st1 = _stats(o1)
    mu1 = st1[0] / NN
    var1 = st1[1] / NN - mu1 * mu1
    isd1 = 1.0 / jnp.sqrt(var1 + 1e-5)

    t2 = _bn_elu_mm(o1, mu1, isd1, g1, be1, Wt2)
    d2 = _bn_elu_mm(o1, mu1, isd1, g1, be1, Wd2)

    o2 = _agg2(t2, d2, pkb, bstart, bcount)

    st2 = _stats(o2)
    mu2 = st2[0] / NN
    var2 = st2[1] / NN - mu2 * mu2
    isd2 = 1.0 / jnp.sqrt(var2 + 1e-5)
    out = _bn_elu(o2, mu2, isd2, g2, be2)
    return out[:NN]


# speedup vs baseline: 2.8825x; 1.0006x over previous
import functools

import jax
import jax.numpy as jnp
from jax import lax
from jax.experimental import pallas as pl
from jax.experimental.pallas import tpu as pltpu
from jax.experimental.pallas import tpu_sc as plsc

NN = 100000
EE = 3200000
NC, NS, LL = 2, 16, 16
NW = NC * NS
BKT_SHIFT = 7
BKT_N = 128
NB = (NN + BKT_N - 1) // BKT_N
NBP = 800
N_PAD = NB * BKT_N
EC = EE // NW
EBP = EE + NBP * 64 + 4096
DUMP = EBP - 1
ROW1, ROW2 = 384, 128
NEG = 0.2

_mesh = lambda: plsc.VectorSubcoreMesh(core_axis_name="c", subcore_axis_name="s")
_SC_PARAMS = pltpu.CompilerParams(needs_layout_passes=False)


def _wid():
    return lax.axis_index("s") * NC + lax.axis_index("c")


_DNUMS = lax.GatherDimensionNumbers(offset_dims=(),
                                    collapsed_slice_dims=(0,),
                                    start_index_map=(0,))


def _splat(vec, lane):
    idx = jnp.full((16, 1), lane, dtype=jnp.int32)
    return lax.gather(vec, idx, _DNUMS, (1,),
                      mode=lax.GatherScatterMode.PROMISE_IN_BOUNDS)


HCH = 10000


@functools.partial(
    pl.kernel,
    out_type=jax.ShapeDtypeStruct((NW, NBP * 16), jnp.int32),
    mesh=_mesh(),
    compiler_params=_SC_PARAMS,
    scratch_types=[pltpu.VMEM((HCH,), jnp.int32),
                   pltpu.VMEM((NBP * 16,), jnp.int32)],
)
def _hist_k(dst, counts_out, dbuf, cnt):
    w = _wid()
    base = pl.multiple_of(w * EC, 8)
    zero16 = jnp.zeros((16,), jnp.int32)

    def z(i, _):
        cnt[pl.ds(i * 16, 16)] = zero16
        return 0
    lax.fori_loop(0, NBP, z, 0)

    iota = lax.iota(jnp.int32, 16)
    one = jnp.ones((16,), jnp.int32)

    def chunk(ci, _):
        pltpu.sync_copy(dst.at[pl.ds(base + ci * HCH, HCH)], dbuf)

        def vec(vi, _):
            d = dbuf[pl.ds(vi * 16, 16)]
            ix = ((d >> BKT_SHIFT) << 4) | iota
            c = plsc.load_gather(cnt, [ix])
            plsc.store_scatter(cnt, [ix], c + one)
            return 0
        lax.fori_loop(0, HCH // 16, vec, 0)
        return 0
    lax.fori_loop(0, EC // HCH, chunk, 0)
    pltpu.sync_copy(cnt, counts_out.at[w])


def _prefix_body(cnt_ref, tri512_ref, tri400_ref, offs_ref, bs_ref, bc_ref):
    c = cnt_ref[...].reshape(NW, NBP, 16)
    c2 = jnp.concatenate([c[t] for t in range(NW)], axis=1)
    c2f = c2.astype(jnp.float32)
    inc = jnp.dot(c2f, tri512_ref[...],
                  preferred_element_type=jnp.float32).astype(jnp.int32)
    tot = inc[:, -1]
    sub = inc - c2
    reg = ((tot + 63) >> 6) << 6
    sinc = jnp.dot(reg.reshape(1, NBP).astype(jnp.float32), tri400_ref[...],
                   preferred_element_type=jnp.float32
                   ).astype(jnp.int32).reshape(NBP)
    starts = sinc - reg
    offs2 = starts[:, None] + sub
    offs = jnp.stack([offs2[:, t * 16:(t + 1) * 16] for t in range(NW)],
                     axis=0)
    offs_ref[...] = offs.reshape(NW, NBP * 16)
    bs_ref[...] = starts.reshape(1, NBP)
    bc_ref[...] = tot.reshape(1, NBP)


def _prefix(counts):
    tri512 = (jnp.arange(512)[:, None] <= jnp.arange(512)[None, :]
              ).astype(jnp.float32)
    tri400 = (jnp.arange(NBP)[:, None] <= jnp.arange(NBP)[None, :]
              ).astype(jnp.float32)
    return pl.pallas_call(
        _prefix_body,
        out_shape=(jax.ShapeDtypeStruct((NW, NBP * 16), jnp.int32),
                   jax.ShapeDtypeStruct((1, NBP), jnp.int32),
                   jax.ShapeDtypeStruct((1, NBP), jnp.int32)),
    )(counts, tri512, tri400)


SCH = 1024
NCH_S = EC // SCH
SCH_T = EC - NCH_S * SCH
SROWS = SCH // 128


@functools.partial(
    pl.kernel,
    out_type=jax.ShapeDtypeStruct((EBP,), jnp.int32),
    mesh=_mesh(),
    compiler_params=_SC_PARAMS,
    scratch_types=[pltpu.VMEM((SCH,), jnp.int32),
                   pltpu.VMEM((SCH,), jnp.int32),
                   pltpu.VMEM((2, SROWS, 128), jnp.int32),
                   pltpu.VMEM((2, SROWS, 128), jnp.int32),
                   pltpu.VMEM((NBP * 16,), jnp.int32),
                   pltpu.SemaphoreType.DMA],
)
def _scatter_k(src, dst, offs, pkb, sch, dch, posb, pkp, offl, sem0):
    w = _wid()
    base = pl.multiple_of(w * EC, 8)
    pltpu.sync_copy(offs.at[w], offl)
    iota = lax.iota(jnp.int32, 16)
    one = jnp.ones((16,), jnp.int32)
    dumpv = jnp.full((16,), DUMP, dtype=jnp.int32)

    def waitset(st):
        for k in range(SROWS):
            pltpu.make_async_copy(pkp.at[st, k], pkb.at[posb.at[st, k]],
                                  sem0).wait()

    def fire(st, nvec):
        def vec(vi, _):
            r = vi >> 3
            m = vi & 7
            d = dch[pl.ds(vi * 16, 16)]
            sv = sch[pl.ds(vi * 16, 16)]
            ix = ((d >> BKT_SHIFT) << 4) | iota
            o = plsc.load_gather(offl, [ix])
            plsc.store_scatter(offl, [ix], o + one)
            posb[st, r, pl.ds(m * 16, 16)] = o
            pkp[st, r, pl.ds(m * 16, 16)] = (
                (sv << BKT_SHIFT) | (d & jnp.int32(BKT_N - 1)))
            return 0
        lax.fori_loop(0, nvec, vec, 0)
        for k in range(SROWS):
            pltpu.make_async_copy(pkp.at[st, k], pkb.at[posb.at[st, k]],
                                  sem0).start()

    def chunk(ci, _):
        st = ci & 1
        pltpu.sync_copy(src.at[pl.ds(base + ci * SCH, SCH)], sch)
        pltpu.sync_copy(dst.at[pl.ds(base + ci * SCH, SCH)], dch)

        @pl.when(ci >= 2)
        def _():
            waitset(st)
        fire(st, SCH // 16)
        return 0
    lax.fori_loop(0, NCH_S, chunk, 0)

    tst = NCH_S & 1
    tbase = pl.multiple_of(base + NCH_S * SCH, 8)
    pltpu.sync_copy(src.at[pl.ds(tbase, SCH_T)], sch.at[pl.ds(0, SCH_T)])
    pltpu.sync_copy(dst.at[pl.ds(tbase, SCH_T)], dch.at[pl.ds(0, SCH_T)])
    waitset(tst)

    def fill(vi, _):
        r = vi >> 3
        m = vi & 7
        posb[tst, r, pl.ds(m * 16, 16)] = dumpv
        return 0
    lax.fori_loop(0, SCH // 16, fill, 0)
    fire(tst, SCH_T // 16)
    waitset(1 - (NCH_S & 1))
    waitset(tst)


def _mm_body(x_ref, w_ref, o_ref):
    o_ref[...] = jnp.dot(x_ref[...], w_ref[...],
                         preferred_element_type=jnp.float32)


def _mm(x, w, n_rows, blk=BKT_N):
    k, m = w.shape
    grid = n_rows // blk
    return pl.pallas_call(
        _mm_body,
        grid=(grid,),
        in_specs=[pl.BlockSpec((blk, k), lambda i: (i, 0)),
                  pl.BlockSpec((k, m), lambda i: (0, 0))],
        out_specs=pl.BlockSpec((blk, m), lambda i: (i, 0)),
        out_shape=jax.ShapeDtypeStruct((n_rows, m), jnp.float32),
    )(x, w)


def _stats_body(x_ref, st_ref):
    blk = x_ref[...]
    s = jnp.sum(blk, axis=0, keepdims=True)
    s2 = jnp.sum(blk * blk, axis=0, keepdims=True)
    st = jnp.concatenate([s, s2], axis=0)

    @pl.when(pl.program_id(0) == 0)
    def _():
        st_ref[...] = st

    @pl.when(pl.program_id(0) > 0)
    def _():
        st_ref[...] = st_ref[...] + st


def _stats(x):
    n, m = x.shape
    return pl.pallas_call(
        _stats_body,
        grid=(n // BKT_N,),
        in_specs=[pl.BlockSpec((BKT_N, m), lambda i: (i, 0))],
        out_specs=pl.BlockSpec((2, m), lambda i: (0, 0)),
        out_shape=jax.ShapeDtypeStruct((2, m), jnp.float32),
    )(x)


def _bn_elu_mm_body(x_ref, mu_ref, isd_ref, g_ref, be_ref, w_ref, o_ref):
    xb = x_ref[...]
    y = g_ref[...] * (xb - mu_ref[...]) * isd_ref[...] + be_ref[...]
    y = jnp.where(y > 0, y, jnp.exp(jnp.minimum(y, 0.0)) - 1.0)
    o_ref[...] = jnp.dot(y, w_ref[...], preferred_element_type=jnp.float32)


def _bn_elu_mm(x, mu, isd, g, be, w):
    n, k = x.shape
    m = w.shape[1]
    return pl.pallas_call(
        _bn_elu_mm_body,
        grid=(n // BKT_N,),
        in_specs=[pl.BlockSpec((BKT_N, k), lambda i: (i, 0)),
                  pl.BlockSpec((1, k), lambda i: (0, 0)),
                  pl.BlockSpec((1, k), lambda i: (0, 0)),
                  pl.BlockSpec((1, k), lambda i: (0, 0)),
                  pl.BlockSpec((1, k), lambda i: (0, 0)),
                  pl.BlockSpec((k, m), lambda i: (0, 0))],
        out_specs=pl.BlockSpec((BKT_N, m), lambda i: (i, 0)),
        out_shape=jax.ShapeDtypeStruct((n, m), jnp.float32),
    )(x, mu.reshape(1, k), isd.reshape(1, k), g.reshape(1, k),
      be.reshape(1, k), w)


def _bn_elu_body(x_ref, mu_ref, isd_ref, g_ref, be_ref, o_ref):
    xb = x_ref[...]
    y = g_ref[...] * (xb - mu_ref[...]) * isd_ref[...] + be_ref[...]
    o_ref[...] = jnp.where(y > 0, y, jnp.exp(jnp.minimum(y, 0.0)) - 1.0)


def _bn_elu(x, mu, isd, g, be):
    n, k = x.shape
    return pl.pallas_call(
        _bn_elu_body,
        grid=(n // BKT_N,),
        in_specs=[pl.BlockSpec((BKT_N, k), lambda i: (i, 0)),
                  pl.BlockSpec((1, k), lambda i: (0, 0)),
                  pl.BlockSpec((1, k), lambda i: (0, 0)),
                  pl.BlockSpec((1, k), lambda i: (0, 0)),
                  pl.BlockSpec((1, k), lambda i: (0, 0))],
        out_specs=pl.BlockSpec((BKT_N, k), lambda i: (i, 0)),
        out_shape=jax.ShapeDtypeStruct((n, k), jnp.float32),
    )(x, mu.reshape(1, k), isd.reshape(1, k), g.reshape(1, k),
      be.reshape(1, k))


ACH = 1024


def _make_agg(row_w, heads, cw, gb):
    acc_w = heads * cw
    a_off = acc_w
    n_qh = cw // 16

    @functools.partial(
        pl.kernel,
        out_type=jax.ShapeDtypeStruct((N_PAD, acc_w), jnp.float32),
        mesh=_mesh(),
        compiler_params=_SC_PARAMS,
        scratch_types=[pltpu.VMEM((BKT_N, acc_w), jnp.float32),
                       pltpu.VMEM((BKT_N, 16), jnp.float32),
                       pltpu.VMEM((BKT_N, 16), jnp.float32),
                       pltpu.VMEM((2, gb, row_w), jnp.float32),
                       pltpu.VMEM((ACH,), jnp.int32),
                       pltpu.VMEM((ACH,), jnp.int32),
                       pltpu.VMEM((1, NBP), jnp.int32),
                       pltpu.VMEM((1, NBP), jnp.int32),
                       pltpu.SemaphoreType.DMA],
    )
    def agg(tbl, dvals, pkb, bs, bc, out, acc, den, dst_a, rows,
            pkc, sidx, bsv, bcv, sem):
        w = _wid()
        pltpu.sync_copy(bs, bsv)
        pltpu.sync_copy(bc, bcv)
        zf = jnp.zeros((16,), jnp.float32)
        eps = jnp.full((16,), 1e-16, dtype=jnp.float32)
        nmax = jnp.full((16,), NN - 1, dtype=jnp.int32)
        zi = jnp.zeros((16,), jnp.int32)
        dmask = jnp.full((16,), BKT_N - 1, dtype=jnp.int32)

        def bucket(bi, _):
            b = bi * NW + w

            @pl.when(b < NB)
            def _():
                nb = bcv[0, pl.ds(b, 16)][0]
                start = pl.multiple_of(bsv[0, pl.ds(b, 16)][0], 8)
                nbase = pl.multiple_of(b * BKT_N, 8)

                @plsc.parallel_loop(0, BKT_N, unroll=4)
                def zrow(r):
                    den[r, pl.ds(0, 16)] = zf
                    dst_a[r, pl.ds(0, 16)] = zf
                    for q in range(acc_w // 16):
                        acc[r, pl.ds(q * 16, 16)] = zf
                pltpu.sync_copy(dvals.at[pl.ds(nbase, BKT_N)], dst_a)

                nch = (nb + ACH - 1) // ACH

                def chunk(ci, _):
                    cbase = pl.multiple_of(start + ci * ACH, 8)
                    pltpu.sync_copy(pkb.at[pl.ds(cbase, ACH)], pkc)

                    @plsc.parallel_loop(0, ACH // 16, unroll=4)
                    def unpk(vi):
                        v = pkc[pl.ds(vi * 16, 16)]
                        sidx[pl.ds(vi * 16, 16)] = jnp.clip(
                            v >> BKT_SHIFT, zi, nmax)
                    nleft = nb - ci * ACH
                    nbat = jnp.minimum(
                        (nleft + gb - 1) // gb, ACH // gb)

                    def gref(k):
                        return sidx.at[pl.ds(k * gb, gb)]
                    pltpu.make_async_copy(tbl.at[gref(0)], rows.at[0],
                                          sem).start()

                    def batch(kb, _):
                        buf = kb & 1

                        @pl.when(kb + 1 < nbat)
                        def _():
                            pltpu.make_async_copy(
                                tbl.at[gref(kb + 1)],
                                rows.at[(kb + 1) & 1], sem).start()
                        pltpu.make_async_copy(tbl.at[gref(kb)],
                                              rows.at[buf], sem).wait()
                        ebase = kb * gb

                        @plsc.parallel_loop(0, gb // 16, unroll=2)
                        def grp(g):
                            dlv = pkc[pl.ds(ebase + g * 16, 16)] & dmask
                            for j in range(16):
                                jb = g * 16 + j
                                dl = dlv[j]
                                valid = (ebase + jb) < nleft
                                dv = dst_a[dl, pl.ds(0, 16)]
                                sv = rows[buf, jb, pl.ds(a_off, 16)]
                                hvs = [rows[buf, jb,
                                            pl.ds(h * cw + q * 16, 16)]
                                       for h in range(heads)
                                       for q in range(n_qh)]
                                a = sv + dv
                                a = jnp.maximum(a, NEG * a)
                                ex = jnp.where(valid, jnp.exp(a), zf)
                                ehs = [_splat(ex, h) for h in range(heads)]
                                for h in range(heads):
                                    for q in range(n_qh):
                                        c0 = h * cw + q * 16
                                        plsc.addupdate(
                                            acc.at[dl, pl.ds(c0, 16)],
                                            ehs[h] * hvs[h * n_qh + q])
                                plsc.addupdate(den.at[dl], ex)
                        return 0
                    lax.fori_loop(0, nbat, batch, 0)
                    return 0
                lax.fori_loop(0, nch, chunk, 0)

                @plsc.parallel_loop(0, BKT_N, unroll=2)
                def nrow(r):
                    dinv = 1.0 / (den[r, pl.ds(0, 16)] + eps)
                    for h in range(heads):
                        eh = _splat(dinv, h)
                        for q in range(n_qh):
                            c0 = h * cw + q * 16
                            acc[r, pl.ds(c0, 16)] = acc[r, pl.ds(c0, 16)] * eh
                pltpu.sync_copy(acc, out.at[pl.ds(nbase, BKT_N)])
            return 0
        lax.fori_loop(0, (NB + NW - 1) // NW, bucket, 0)

    return agg


_agg1 = _make_agg(ROW1, 4, 64, 64)
_agg2 = _make_agg(ROW2, 1, 32, 128)


def _head_mats(a_src, a_dst, heads, cw):
    eye = jnp.eye(heads, dtype=jnp.float32)
    m = (eye[:, None, :] * a_src[:, :, None]).reshape(heads * cw, heads)
    md = (eye[:, None, :] * a_dst[:, :, None]).reshape(heads * cw, heads)
    pad = jnp.zeros((heads * cw, 16 - heads), jnp.float32)
    return (jnp.concatenate([m, pad], axis=1),
            jnp.concatenate([md, pad], axis=1))


def kernel(x, edge_index, W1, a_src1, a_dst1, b1, g1, be1,
           W2, a_src2, a_dst2, b2, g2, be2):
    f32 = jnp.float32
    As1, Ad1 = _head_mats(a_src1, a_dst1, 4, 64)
    As2, Ad2 = _head_mats(a_src2, a_dst2, 1, 32)
    hi = jax.lax.Precision.HIGHEST
    Wt1 = jnp.concatenate(
        [W1, jnp.dot(W1, As1, precision=hi),
         jnp.zeros((22, ROW1 - 272), f32)], axis=1
    ).astype(f32)
    Wd1 = jnp.dot(W1, Ad1, precision=hi).astype(f32)
    Wt2 = jnp.concatenate(
        [W2, jnp.dot(W2, As2, precision=hi),
         jnp.zeros((256, ROW2 - 48), f32)], axis=1
    ).astype(f32)
    Wd2 = jnp.dot(W2, Ad2, precision=hi).astype(f32)

    src_e = edge_index[0]
    dst_e = edge_index[1]
    counts = _hist_k(dst_e)
    offs, bstart, bcount = _prefix(counts)
    pkb = _scatter_k(src_e, dst_e, offs)

    t1 = _mm(x, Wt1, N_PAD)
    d1 = _mm(x, Wd1, N_PAD)

    o1 = _agg1(t1, d1, pkb, bstart, bcount)

    st1 = _stats(o1)
    mu1 = st1[0] / NN
    var1 = st1[1] / NN - mu1 * mu1
    isd1 = 1.0 / jnp.sqrt(var1 + 1e-5)

    t2 = _bn_elu_mm(o1, mu1, isd1, g1, be1, Wt2)
    d2 = _bn_elu_mm(o1, mu1, isd1, g1, be1, Wd2)

    o2 = _agg2(t2, d2, pkb, bstart, bcount)

    st2 = _stats(o2)
    mu2 = st2[0] / NN
    var2 = st2[1] / NN - mu2 * mu2
    isd2 = 1.0 / jnp.sqrt(var2 + 1e-5)
    out = _bn_elu(o2, mu2, isd2, g2, be2)
    return out[:NN]
